# baseline (speedup 1.0000x reference)
import functools

import jax
import jax.numpy as jnp
from jax import lax
from jax.experimental import pallas as pl
from jax.experimental.pallas import tpu as pltpu
from jax.experimental.pallas import tpu_sc as plsc

N_NODES = 10000
D_FEAT = 128
N_EDGES = 320000

NC = 2
NS = 16

CH = 128
PH = (80, 78)
NCHUNK = sum(PH)
E_PAD = NS * NCHUNK * CH

ROWS_PER_TILE = N_NODES // NS
ZROWS = 626
ACC_ROWS = NS * ZROWS
DZROWS = 640
DEG_ROWS = NS * DZROWS

_WB = [(0, 128), (128, 128), (256, 128), (384, 128), (512, 113)]
_DWB = [(0, 128), (128, 128), (256, 128), (384, 128), (512, 128)]


def _sc_accumulate(x, pakp, zacc, zdeg, ones):
    mesh = plsc.VectorSubcoreMesh(core_axis_name="c", subcore_axis_name="s")

    @functools.partial(
        pl.kernel,
        out_type=[
            jax.ShapeDtypeStruct((NS, ROWS_PER_TILE, D_FEAT), jnp.float32),
            jax.ShapeDtypeStruct((1, DEG_ROWS), jnp.float32),
        ],
        mesh=mesh,
        scratch_types=[
            pltpu.VMEM((PH[0], CH), jnp.int32),
            pltpu.VMEM((2, CH), jnp.int32),
            pltpu.VMEM((2, CH), jnp.int32),
            pltpu.VMEM((CH, D_FEAT), jnp.float32),
            pltpu.VMEM((CH, D_FEAT), jnp.float32),
            pltpu.VMEM((CH,), jnp.float32),
            pltpu.VMEM_SHARED((ACC_ROWS, D_FEAT), jnp.float32),
            pltpu.VMEM_SHARED((DEG_ROWS,), jnp.float32),
            pltpu.SemaphoreType.DMA,
            pltpu.SemaphoreType.DMA,
            pltpu.SemaphoreType.DMA,
            pltpu.SemaphoreType.DMA,
            pltpu.SemaphoreType.DMA,
        ],
    )
    def k(x_hbm, pak_hbm, zacc_hbm, zdeg_hbm, one_hbm,
          psum_hbm, pdeg_hbm,
          ipak_v, icolc, irowc, rows_a, rows_b, ones_v, acc_sh, deg_sh,
          g_sem_a, g_sem_b, s_sem_a, s_sem_b, d_sem):
        cid = lax.axis_index("c")
        sid = lax.axis_index("s")

        rows = (rows_a, rows_b)
        g_sem = (g_sem_a, g_sem_b)
        s_sem = (s_sem_a, s_sem_b)

        @pl.when(cid == 0)
        def _core0():
            pltpu.sync_copy(zacc_hbm, acc_sh.at[pl.ds(sid * ZROWS, ZROWS)])
            pltpu.sync_copy(zdeg_hbm, deg_sh.at[pl.ds(sid * DZROWS, DZROWS)])
            pltpu.sync_copy(one_hbm, ones_v)

            plsc.subcore_barrier()

            def unpack(j, s):
                for jj in range(CH // 16):
                    v = ipak_v[j, pl.ds(jj * 16, 16)]
                    irowc[s, pl.ds(jj * 16, 16)] = lax.shift_right_logical(
                        v, 14)
                    icolc[s, pl.ds(jj * 16, 16)] = lax.bitwise_and(v, 16383)

            def start_gather(s):
                pltpu.async_copy(x_hbm.at[icolc.at[s]], rows[s], g_sem[s])

            def wait_gather(s):
                pltpu.make_async_copy(x_hbm.at[icolc.at[s]], rows[s],
                                      g_sem[s]).wait()

            def half(j, s, nch):
                wait_gather(s)
                pltpu.async_copy(rows[s], acc_sh.at[irowc.at[s]], s_sem[s],
                                 add=True)
                pltpu.async_copy(ones_v, deg_sh.at[irowc.at[s]], d_sem,
                                 add=True)
                pltpu.make_async_copy(rows[s], acc_sh.at[irowc.at[s]],
                                      s_sem[s]).wait()
                pltpu.make_async_copy(ones_v, deg_sh.at[irowc.at[s]],
                                      d_sem).wait()
                jn = j + 2

                @pl.when(jn < nch)
                def _():
                    unpack(jn, s)
                    start_gather(s)

            base = 0
            for ph_n in PH:
                pltpu.sync_copy(pak_hbm.at[sid, pl.ds(base, ph_n)],
                                ipak_v.at[pl.ds(0, ph_n)])
                unpack(0, 0)
                start_gather(0)
                unpack(1, 1)
                start_gather(1)

                def body(kk, c, n=ph_n):
                    half(2 * kk, 0, n)
                    half(2 * kk + 1, 1, n)
                    return c

                lax.fori_loop(0, ph_n // 2, body, 0)
                base += ph_n

            plsc.subcore_barrier()

            nbase = sid * ROWS_PER_TILE
            for off, sz in _WB:
                pltpu.sync_copy(acc_sh.at[pl.ds(nbase + off, sz)],
                                rows_a.at[pl.ds(0, sz)])
                pltpu.sync_copy(rows_a.at[pl.ds(0, sz)],
                                psum_hbm.at[sid, pl.ds(off, sz)])
            dbase = sid * DZROWS
            for off, sz in _DWB:
                pltpu.sync_copy(deg_sh.at[pl.ds(dbase + off, sz)],
                                ones_v.at[pl.ds(0, sz)])
                pltpu.sync_copy(ones_v.at[pl.ds(0, sz)],
                                pdeg_hbm.at[0, pl.ds(dbase + off, sz)])

    return k(x, pakp, zacc, zdeg, ones)


def _combine_body(s_ref, d_ref, o_ref):
    d = d_ref[...]
    o_ref[...] = jnp.where(d > 0.0, s_ref[...] / jnp.maximum(d, 1.0), 0.0)


def _combine(psum, pdeg):
    r = 2000
    g = N_NODES // r
    return pl.pallas_call(
        _combine_body,
        grid=(g,),
        in_specs=[
            pl.BlockSpec((r, D_FEAT), lambda i: (i, 0)),
            pl.BlockSpec((r, 1), lambda i: (i, 0)),
        ],
        out_specs=pl.BlockSpec((r, D_FEAT), lambda i: (i, 0)),
        out_shape=jax.ShapeDtypeStruct((N_NODES, D_FEAT), jnp.float32),
    )(psum, pdeg)


def kernel(x, edge_index):
    row = edge_index[0].astype(jnp.int32)
    col = edge_index[1].astype(jnp.int32)
    npad = E_PAD - N_EDGES
    pak = jnp.left_shift(row, 14) | col
    pakp = jnp.concatenate(
        [pak, jnp.full((npad,), N_NODES << 14, jnp.int32)]
    ).reshape(NS, NCHUNK, CH)
    zacc = jnp.zeros((ZROWS, D_FEAT), jnp.float32)
    zdeg = jnp.zeros((DZROWS,), jnp.float32)
    ones = jnp.ones((CH,), jnp.float32)
    psum, pdeg = _sc_accumulate(x, pakp, zacc, zdeg, ones)
    return _combine(psum.reshape(N_NODES, D_FEAT),
                    pdeg.reshape(DEG_ROWS)[:N_NODES, None])


# speedup vs baseline: 1.2759x; 1.2759x over previous
import functools

import jax
import jax.numpy as jnp
from jax import lax
from jax.experimental import pallas as pl
from jax.experimental.pallas import tpu as pltpu
from jax.experimental.pallas import tpu_sc as plsc

N_NODES = 10000
D_FEAT = 128
N_EDGES = 320000

NC = 2
NS = 16
NW = NC * NS

CH = 128
NCH0 = 112
NCH1 = 46
E_PAD = NS * (NCH0 + NCH1) * CH

ROWS_PER_TILE = N_NODES // NS
ZROWS = 626
ACC_ROWS = NS * ZROWS
DZROWS = 640
DEG_ROWS = NS * DZROWS

_WB = [(0, 128), (128, 128), (256, 128), (384, 128), (512, 113)]
_DWB = [(0, 128), (128, 128), (256, 128), (384, 128), (512, 128)]


def _sc_accumulate(x, pakp, zacc, zdeg, ones):
    mesh = plsc.VectorSubcoreMesh(core_axis_name="c", subcore_axis_name="s")

    @functools.partial(
        pl.kernel,
        out_type=[
            jax.ShapeDtypeStruct((NC, NS, ROWS_PER_TILE, D_FEAT), jnp.float32),
            jax.ShapeDtypeStruct((NC, 1, DEG_ROWS), jnp.float32),
        ],
        mesh=mesh,
        scratch_types=[
            pltpu.VMEM((NCH0, CH), jnp.int32),
            pltpu.VMEM((2, CH), jnp.int32),
            pltpu.VMEM((2, CH), jnp.int32),
            pltpu.VMEM((CH, D_FEAT), jnp.float32),
            pltpu.VMEM((CH, D_FEAT), jnp.float32),
            pltpu.VMEM((CH,), jnp.float32),
            pltpu.VMEM_SHARED((ACC_ROWS, D_FEAT), jnp.float32),
            pltpu.VMEM_SHARED((DEG_ROWS,), jnp.float32),
            pltpu.SemaphoreType.DMA,
            pltpu.SemaphoreType.DMA,
            pltpu.SemaphoreType.DMA,
            pltpu.SemaphoreType.DMA,
            pltpu.SemaphoreType.DMA,
        ],
    )
    def k(x_hbm, pak_hbm, zacc_hbm, zdeg_hbm, one_hbm,
          psum_hbm, pdeg_hbm,
          ipak_v, icolc, irowc, rows_a, rows_b, ones_v, acc_sh, deg_sh,
          g_sem_a, g_sem_b, s_sem_a, s_sem_b, d_sem):
        cid = lax.axis_index("c")
        sid = lax.axis_index("s")
        wid = cid * NS + sid

        rows = (rows_a, rows_b)
        g_sem = (g_sem_a, g_sem_b)
        s_sem = (s_sem_a, s_sem_b)

        pltpu.sync_copy(zacc_hbm, acc_sh.at[pl.ds(sid * ZROWS, ZROWS)])
        pltpu.sync_copy(zdeg_hbm, deg_sh.at[pl.ds(sid * DZROWS, DZROWS)])

        pltpu.sync_copy(one_hbm, ones_v)
        pltpu.sync_copy(pak_hbm.at[wid], ipak_v)

        plsc.subcore_barrier()

        def unpack(j, s):
            for jj in range(CH // 16):
                v = ipak_v[j, pl.ds(jj * 16, 16)]
                irowc[s, pl.ds(jj * 16, 16)] = lax.shift_right_logical(v, 14)
                icolc[s, pl.ds(jj * 16, 16)] = lax.bitwise_and(v, 16383)

        def start_gather(s):
            pltpu.async_copy(x_hbm.at[icolc.at[s]], rows[s], g_sem[s])

        def wait_gather(s):
            pltpu.make_async_copy(x_hbm.at[icolc.at[s]], rows[s],
                                  g_sem[s]).wait()

        unpack(0, 0)
        start_gather(0)
        unpack(1, 1)
        start_gather(1)

        nchunk = jnp.where(cid == 0, NCH0, NCH1)

        def half(j, s):
            wait_gather(s)
            pltpu.async_copy(rows[s], acc_sh.at[irowc.at[s]], s_sem[s],
                             add=True)
            pltpu.async_copy(ones_v, deg_sh.at[irowc.at[s]], d_sem, add=True)
            pltpu.make_async_copy(rows[s], acc_sh.at[irowc.at[s]],
                                  s_sem[s]).wait()
            pltpu.make_async_copy(ones_v, deg_sh.at[irowc.at[s]],
                                  d_sem).wait()
            jn = j + 2

            @pl.when(jn < nchunk)
            def _():
                unpack(jn, s)
                start_gather(s)

        def body(kk, c):
            half(2 * kk, 0)
            half(2 * kk + 1, 1)
            return c

        lax.fori_loop(0, nchunk // 2, body, 0)

        plsc.subcore_barrier()

        nbase = sid * ROWS_PER_TILE
        for off, sz in _WB:
            pltpu.sync_copy(acc_sh.at[pl.ds(nbase + off, sz)],
                            rows_a.at[pl.ds(0, sz)])
            pltpu.sync_copy(rows_a.at[pl.ds(0, sz)],
                            psum_hbm.at[cid, sid, pl.ds(off, sz)])
        dbase = sid * DZROWS
        for off, sz in _DWB:
            pltpu.sync_copy(deg_sh.at[pl.ds(dbase + off, sz)],
                            ones_v.at[pl.ds(0, sz)])
            pltpu.sync_copy(ones_v.at[pl.ds(0, sz)],
                            pdeg_hbm.at[cid, 0, pl.ds(dbase + off, sz)])

    return k(x, pakp, zacc, zdeg, ones)


def _combine_body(s_ref, d_ref, o_ref):
    s = s_ref[0] + s_ref[1]
    d = d_ref[0] + d_ref[1]
    o_ref[...] = jnp.where(d > 0.0, s / jnp.maximum(d, 1.0), 0.0)


def _combine(psum, pdeg):
    r = 2000
    g = N_NODES // r
    return pl.pallas_call(
        _combine_body,
        grid=(g,),
        in_specs=[
            pl.BlockSpec((2, r, D_FEAT), lambda i: (0, i, 0)),
            pl.BlockSpec((2, r, 1), lambda i: (0, i, 0)),
        ],
        out_specs=pl.BlockSpec((r, D_FEAT), lambda i: (i, 0)),
        out_shape=jax.ShapeDtypeStruct((N_NODES, D_FEAT), jnp.float32),
    )(psum, pdeg)


def kernel(x, edge_index):
    row = edge_index[0].astype(jnp.int32)
    col = edge_index[1].astype(jnp.int32)
    npad = E_PAD - N_EDGES
    pak = jnp.left_shift(row, 14) | col
    pakf = jnp.concatenate(
        [pak, jnp.full((npad,), N_NODES << 14, jnp.int32)])
    n0 = NS * NCH0 * CH
    pak0 = pakf[:n0].reshape(NS, NCH0, CH)
    pak1 = pakf[n0:].reshape(NS, NCH1, CH)
    pak1 = jnp.pad(pak1, ((0, 0), (0, NCH0 - NCH1), (0, 0)),
                   constant_values=N_NODES << 14)
    pakp = jnp.concatenate([pak0, pak1], axis=0)
    zacc = jnp.zeros((ZROWS, D_FEAT), jnp.float32)
    zdeg = jnp.zeros((DZROWS,), jnp.float32)
    ones = jnp.ones((CH,), jnp.float32)
    psum, pdeg = _sc_accumulate(x, pakp, zacc, zdeg, ones)
    return _combine(psum.reshape(NC, N_NODES, D_FEAT),
                    pdeg.reshape(NC, DEG_ROWS)[:, :N_NODES, None])
